# baseline (speedup 1.0000x reference)
import functools

import jax
import jax.numpy as jnp
from jax import lax
from jax.experimental import pallas as pl
from jax.experimental.pallas import tpu as pltpu
from jax.experimental.pallas import tpu_sc as plsc

N_NODE = 10000
D = 128
L = 16
NC, NS = 2, 16
NW = NC * NS
EW = 10240
E_PAD = NW * EW
SUPER = 1280
NSUP = EW // SUPER
CH = 32
NCH = SUPER // CH
NP = NCH // 2
GR = CH // L
N_PAD = 10240
RW = N_PAD // NS
ZR = 16
NZ = RW // ZR
PAD_ROWS = 10048


def _tables_body(hid_ref, rel_ref, ws_ref, wr_ref, node_ref, relo_ref):
    h = hid_ref[...]
    r = rel_ref[...]
    node_ref[:, :D] = h
    node_ref[:, D:] = lax.dot_general(
        h, ws_ref[...], (((1,), (1,)), ((), ())),
        preferred_element_type=jnp.float32)
    relo_ref[:, :D] = r
    relo_ref[:, D:] = lax.dot_general(
        r, wr_ref[...], (((1,), (1,)), ((), ())),
        preferred_element_type=jnp.float32)


def _build_tables(hid_p, rel_p, ws, wr):
    nblk = 8
    rows = PAD_ROWS // nblk
    return pl.pallas_call(
        _tables_body,
        grid=(nblk,),
        in_specs=[
            pl.BlockSpec((rows, D), lambda i: (i, 0)),
            pl.BlockSpec((rows, D), lambda i: (i, 0)),
            pl.BlockSpec((D, D), lambda i: (0, 0)),
            pl.BlockSpec((D, D), lambda i: (0, 0)),
        ],
        out_specs=[
            pl.BlockSpec((rows, 2 * D), lambda i: (i, 0)),
            pl.BlockSpec((rows, 2 * D), lambda i: (i, 0)),
        ],
        out_shape=[
            jax.ShapeDtypeStruct((PAD_ROWS, 2 * D), jnp.float32),
            jax.ShapeDtypeStruct((PAD_ROWS, 2 * D), jnp.float32),
        ],
    )(hid_p, rel_p, ws, wr)


def _c4_body(kg_ref, wkg_ref, wb_ref, out_ref):
    kg = kg_ref[...]
    w1 = wkg_ref[:, :D]
    w2 = wkg_ref[:, D:]
    kg1 = lax.dot_general(kg, w1, (((1,), (1,)), ((), ())),
                          preferred_element_type=jnp.float32)
    kg2 = lax.dot_general(kg, w2, (((1,), (1,)), ((), ())),
                          preferred_element_type=jnp.float32)
    c = kg1[:, None, :] + kg2[None, :, :] + wb_ref[...][None, None, :]
    c = c.reshape(4, D)
    out_ref[...] = jnp.concatenate([c, jnp.zeros((4, D), jnp.float32)], axis=0)


def _build_c4(kgemb, wkg_w, wkg_b):
    return pl.pallas_call(
        _c4_body,
        out_shape=jax.ShapeDtypeStruct((8, D), jnp.float32),
    )(kgemb, wkg_w, wkg_b)


def _lane_sum(v):
    lanes = lax.iota(jnp.int32, L)
    dnums = lax.GatherDimensionNumbers(
        offset_dims=(), collapsed_slice_dims=(0,), start_index_map=(0,))
    for sh in (1, 2, 4, 8):
        perm = lax.bitwise_xor(lanes, jnp.full((L,), sh, jnp.int32))
        v = v + lax.gather(v, perm[:, None], dnums, slice_sizes=(1,),
                           mode=lax.GatherScatterMode.PROMISE_IN_BOUNDS)
    return v


def _sc_body(node_hbm, rel_hbm, c4_hbm, w_hbm, b0_hbm, ekg_hbm,
             erl_hbm, esb_hbm, eob_hbm,
             out_hbm, acc_sh, nrows0, nrows1, rrows0, rrows1, msg0,
             zbuf, mkg, mrl, msb, mob, subi0, subi1, reli0, reli1, obji0,
             obji1, c4v, wv, b0v, semn0, semr0, semn1, semr1):
    cid = lax.axis_index("c")
    sid = lax.axis_index("s")
    wid = cid * NS + sid

    pltpu.sync_copy(c4_hbm, c4v)
    pltpu.sync_copy(w_hbm, wv)
    pltpu.sync_copy(b0_hbm, b0v)

    zero = jnp.zeros((L,), jnp.float32)

    def _zrow(r, carry):
        for k in range(8):
            zbuf[r, pl.ds(k * L, L)] = zero
        return carry

    lax.fori_loop(0, ZR, _zrow, 0)
    for i in range(NZ):
        pltpu.sync_copy(zbuf, acc_sh.at[pl.ds(sid * RW + i * ZR, ZR)])
    plsc.subcore_barrier()

    wk = tuple(wv[pl.ds(k * L, L)] for k in range(8))
    b0 = b0v[...]
    kconst = tuple(lax.iota(jnp.int32, L) + jnp.full((L,), k * L, jnp.int32)
                   for k in range(8))

    ebase = wid * EW
    slots = ((nrows0, rrows0, subi0, reli0, obji0, semn0, semr0, msg0),
             (nrows1, rrows1, subi1, reli1, obji1, semn1, semr1, msg0))

    def _prep(c, slot):
        su, re_, ob = slots[slot][2:5]
        for g in range(GR):
            su[0, 0, pl.ds(g * L, L)] = msb[pl.ds(c * CH + g * L, L)]
            re_[0, 0, pl.ds(g * L, L)] = mrl[pl.ds(c * CH + g * L, L)]
            ob[0, 0, pl.ds(g * L, L)] = mob[pl.ds(c * CH + g * L, L)]

    def _fire(slot):
        nr, rr, su, re_, _, sn, sr = slots[slot][:7]
        pltpu.async_copy(node_hbm.at[su.at[0, 0]], nr, sn)
        pltpu.async_copy(rel_hbm.at[re_.at[0, 0]], rr, sr)

    def _wait(slot):
        nr, rr, su, re_, _, sn, sr = slots[slot][:7]
        pltpu.make_async_copy(node_hbm.at[su.at[0, 0]], nr, sn).wait()
        pltpu.make_async_copy(rel_hbm.at[re_.at[0, 0]], rr, sr).wait()

    lane_dn = lax.GatherDimensionNumbers(
        offset_dims=(), collapsed_slice_dims=(0,), start_index_map=(0,))

    def _compute_scatter(c, slot):
        nr, rr, _, _, ob, _, _ = slots[slot][:7]
        ms = slots[slot][7]
        for g in range(GR):
            kv = mkg[pl.ds(c * CH + g * L, L)] * jnp.full((L,), D, jnp.int32)

            def _oct(q, carry):
                for u in range(8):
                    j2 = q * 8 + u
                    j = g * L + j2
                    perm = lax.broadcast(j2, (L,))
                    kgs = lax.gather(kv, perm[:, None], lane_dn,
                                     slice_sizes=(1,),
                                     mode=lax.GatherScatterMode.PROMISE_IN_BOUNDS)
                    acc = jnp.zeros((L,), jnp.float32)
                    for k in range(8):
                        a = nr[j, pl.ds(D + k * L, L)]
                        b = rr[j, pl.ds(D + k * L, L)]
                        cc = plsc.load_gather(c4v, [kgs + kconst[k]])
                        acc = acc + jnp.maximum(a + b + cc, 0.0) * wk[k]
                    sv = _lane_sum(acc) + b0
                    av = 1.0 / (1.0 + jnp.exp(-sv))
                    for k in range(8):
                        m = (nr[j, pl.ds(k * L, L)]
                             + rr[j, pl.ds(k * L, L)]) * av
                        ms[j, pl.ds(k * L, L)] = m
                return carry

            lax.fori_loop(0, 2, _oct, 0)
        pltpu.sync_copy(ms, acc_sh.at[ob.at[0, 0]], add=True)

    def _super(s_i, carry):
        sb = ebase + s_i * SUPER
        for col, buf in ((ekg_hbm, mkg), (erl_hbm, mrl), (esb_hbm, msb),
                         (eob_hbm, mob)):
            pltpu.sync_copy(col.at[pl.ds(sb, SUPER)], buf)

        _prep(0, 0)
        _fire(0)
        _prep(1, 1)
        _fire(1)

        def _pair(c2, carry2):
            c0 = c2 * 2
            _wait(0)
            _compute_scatter(c0, 0)

            def _pf0():
                _prep(c0 + 2, 0)
                _fire(0)

            pl.when(c2 < NP - 1)(_pf0)
            _wait(1)
            _compute_scatter(c0 + 1, 1)

            def _pf1():
                _prep(c0 + 3, 1)
                _fire(1)

            pl.when(c2 < NP - 1)(_pf1)
            return carry2

        lax.fori_loop(0, NP, _pair, 0)
        return carry

    lax.fori_loop(0, NSUP, _super, 0)
    plsc.subcore_barrier()

    for i in range(NZ):
        r0 = sid * RW + i * ZR
        pltpu.sync_copy(acc_sh.at[pl.ds(r0, ZR)], zbuf)
        pltpu.sync_copy(zbuf, out_hbm.at[cid, pl.ds(r0, ZR)])


def _sc_edges(node_tab, rel_tab, c4, wvec, b0vec, ecols):
    ekg, erl, esb, eob = ecols
    mesh = plsc.VectorSubcoreMesh(core_axis_name="c", subcore_axis_name="s",
                                  num_cores=NC, num_subcores=NS)
    fn = pl.kernel(
        _sc_body,
        out_type=jax.ShapeDtypeStruct((NC, N_PAD, D), jnp.float32),
        mesh=mesh,
        compiler_params=pltpu.CompilerParams(needs_layout_passes=False),
        scratch_types=[
            pltpu.VMEM_SHARED((N_PAD, D), jnp.float32),
            pltpu.VMEM((CH, 2 * D), jnp.float32),
            pltpu.VMEM((CH, 2 * D), jnp.float32),
            pltpu.VMEM((CH, 2 * D), jnp.float32),
            pltpu.VMEM((CH, 2 * D), jnp.float32),
            pltpu.VMEM((CH, D), jnp.float32),
            pltpu.VMEM((ZR, D), jnp.float32),
            pltpu.VMEM((SUPER,), jnp.int32),
            pltpu.VMEM((SUPER,), jnp.int32),
            pltpu.VMEM((SUPER,), jnp.int32),
            pltpu.VMEM((SUPER,), jnp.int32),
            pltpu.VMEM((1, 1, CH), jnp.int32),
            pltpu.VMEM((1, 1, CH), jnp.int32),
            pltpu.VMEM((1, 1, CH), jnp.int32),
            pltpu.VMEM((1, 1, CH), jnp.int32),
            pltpu.VMEM((1, 1, CH), jnp.int32),
            pltpu.VMEM((1, 1, CH), jnp.int32),
            pltpu.VMEM((4 * D,), jnp.float32),
            pltpu.VMEM((D,), jnp.float32),
            pltpu.VMEM((L,), jnp.float32),
            pltpu.SemaphoreType.DMA,
            pltpu.SemaphoreType.DMA,
            pltpu.SemaphoreType.DMA,
            pltpu.SemaphoreType.DMA,
        ],
    )
    return fn(node_tab, rel_tab, c4, wvec, b0vec, ekg, erl, esb, eob)


def _fin_body(p_ref, wh_ref, out_ref):
    s = p_ref[0] + p_ref[1]
    out_ref[...] = lax.dot_general(
        s, wh_ref[...], (((1,), (1,)), ((), ())),
        preferred_element_type=jnp.float32)


def _finish(partials, wh):
    nblk = 8
    rows = N_PAD // nblk
    return pl.pallas_call(
        _fin_body,
        grid=(nblk,),
        in_specs=[
            pl.BlockSpec((NC, rows, D), lambda i: (0, i, 0)),
            pl.BlockSpec((D, D), lambda i: (0, 0)),
        ],
        out_specs=pl.BlockSpec((rows, D), lambda i: (i, 0)),
        out_shape=jax.ShapeDtypeStruct((N_PAD, D), jnp.float32),
    )(partials, wh)


def kernel(hidden, edges, n_node, kgemb, left_num, rela_embed, Ws, Wr,
           Wkg_W, Wkg_b, walpha_W, walpha_b, Wh):
    hid_p = jnp.pad(hidden, ((0, PAD_ROWS - hidden.shape[0]), (0, 0)))
    rel_p = jnp.pad(rela_embed, ((0, PAD_ROWS - rela_embed.shape[0]), (0, 0)))
    e32 = edges.astype(jnp.int32)
    npad = E_PAD - e32.shape[0]
    objc = jnp.minimum(e32[:, 5], hidden.shape[0] - 1)
    objc = jnp.pad(objc, (0, npad), constant_values=N_NODE)
    kidx = 2 * (e32[:, 1] >= left_num).astype(jnp.int32) \
        + (e32[:, 3] >= left_num).astype(jnp.int32)
    ecols = (jnp.pad(kidx, (0, npad)), jnp.pad(e32[:, 2], (0, npad)),
             jnp.pad(e32[:, 4], (0, npad)), objc)
    wvec = walpha_W.reshape(D)
    b0vec = jnp.broadcast_to(walpha_b.reshape(1), (L,)).astype(jnp.float32)

    node_tab, rel_tab = _build_tables(hid_p, rel_p, Ws, Wr)
    c4 = _build_c4(kgemb, Wkg_W, Wkg_b)[:4].reshape(4 * D)
    partials = _sc_edges(node_tab, rel_tab, c4, wvec, b0vec, ecols)
    return _finish(partials, Wh)[:N_NODE]


# speedup vs baseline: 1.2248x; 1.2248x over previous
import functools

import jax
import jax.numpy as jnp
from jax import lax
from jax.experimental import pallas as pl
from jax.experimental.pallas import tpu as pltpu
from jax.experimental.pallas import tpu_sc as plsc

N_NODE = 10000
D = 128
L = 16
NC, NS = 2, 16
NW = NC * NS
EW = 10240
E_PAD = NW * EW
SUPER = 1280
NSUP = EW // SUPER
CH = 32
NCH = SUPER // CH
NP = NCH // 2
GR = CH // L
N_PAD = 10240
RW = N_PAD // NS
ZR = 16
NZ = RW // ZR
PAD_ROWS = 10048


def _tables_body(hid_ref, rel_ref, ws_ref, wr_ref, node_ref, relo_ref):
    h = hid_ref[...]
    r = rel_ref[...]
    node_ref[:, :D] = h
    node_ref[:, D:] = lax.dot_general(
        h, ws_ref[...], (((1,), (1,)), ((), ())),
        preferred_element_type=jnp.float32)
    relo_ref[:, :D] = r
    relo_ref[:, D:] = lax.dot_general(
        r, wr_ref[...], (((1,), (1,)), ((), ())),
        preferred_element_type=jnp.float32)


def _build_tables(hid_p, rel_p, ws, wr):
    nblk = 8
    rows = PAD_ROWS // nblk
    return pl.pallas_call(
        _tables_body,
        grid=(nblk,),
        in_specs=[
            pl.BlockSpec((rows, D), lambda i: (i, 0)),
            pl.BlockSpec((rows, D), lambda i: (i, 0)),
            pl.BlockSpec((D, D), lambda i: (0, 0)),
            pl.BlockSpec((D, D), lambda i: (0, 0)),
        ],
        out_specs=[
            pl.BlockSpec((rows, 2 * D), lambda i: (i, 0)),
            pl.BlockSpec((rows, 2 * D), lambda i: (i, 0)),
        ],
        out_shape=[
            jax.ShapeDtypeStruct((PAD_ROWS, 2 * D), jnp.float32),
            jax.ShapeDtypeStruct((PAD_ROWS, 2 * D), jnp.float32),
        ],
    )(hid_p, rel_p, ws, wr)


def _c4_body(kg_ref, wkg_ref, wb_ref, out_ref):
    kg = kg_ref[...]
    w1 = wkg_ref[:, :D]
    w2 = wkg_ref[:, D:]
    kg1 = lax.dot_general(kg, w1, (((1,), (1,)), ((), ())),
                          preferred_element_type=jnp.float32)
    kg2 = lax.dot_general(kg, w2, (((1,), (1,)), ((), ())),
                          preferred_element_type=jnp.float32)
    c = kg1[:, None, :] + kg2[None, :, :] + wb_ref[...][None, None, :]
    c = c.reshape(4, D)
    out_ref[...] = jnp.concatenate([c, jnp.zeros((4, D), jnp.float32)], axis=0)


def _build_c4(kgemb, wkg_w, wkg_b):
    return pl.pallas_call(
        _c4_body,
        out_shape=jax.ShapeDtypeStruct((8, D), jnp.float32),
    )(kgemb, wkg_w, wkg_b)


def _lane_sum(v):
    lanes = lax.iota(jnp.int32, L)
    dnums = lax.GatherDimensionNumbers(
        offset_dims=(), collapsed_slice_dims=(0,), start_index_map=(0,))
    for sh in (1, 2, 4, 8):
        perm = lax.bitwise_xor(lanes, jnp.full((L,), sh, jnp.int32))
        v = v + lax.gather(v, perm[:, None], dnums, slice_sizes=(1,),
                           mode=lax.GatherScatterMode.PROMISE_IN_BOUNDS)
    return v


def _sc_body(node_hbm, rel_hbm, c4_hbm, w_hbm, b0_hbm, ekg_hbm,
             erl_hbm, esb_hbm, eob_hbm,
             out_hbm, acc_sh, nrows0, nrows1, rrows0, rrows1, msg0,
             zbuf, mkg, mrl, msb, mob, subi0, subi1, reli0, reli1, obji0,
             obji1, c4v, wv, b0v, semn0, semr0, semn1, semr1):
    cid = lax.axis_index("c")
    sid = lax.axis_index("s")
    wid = cid * NS + sid

    pltpu.sync_copy(c4_hbm, c4v)
    pltpu.sync_copy(w_hbm, wv)
    pltpu.sync_copy(b0_hbm, b0v)

    zero = jnp.zeros((L,), jnp.float32)

    def _zrow(r, carry):
        for k in range(8):
            zbuf[r, pl.ds(k * L, L)] = zero
        return carry

    lax.fori_loop(0, ZR, _zrow, 0)
    for i in range(NZ):
        pltpu.sync_copy(zbuf, acc_sh.at[pl.ds(sid * RW + i * ZR, ZR)])
    plsc.subcore_barrier()

    wk = tuple(wv[pl.ds(k * L, L)] for k in range(8))
    b0 = b0v[...]
    kconst = tuple(lax.iota(jnp.int32, L) + jnp.full((L,), k * L, jnp.int32)
                   for k in range(8))

    ebase = wid * EW
    slots = ((nrows0, rrows0, subi0, reli0, obji0, semn0, semr0, msg0),
             (nrows1, rrows1, subi1, reli1, obji1, semn1, semr1, msg0))

    def _prep(c, slot):
        su, re_, ob = slots[slot][2:5]
        for g in range(GR):
            su[0, 0, pl.ds(g * L, L)] = msb[pl.ds(c * CH + g * L, L)]
            re_[0, 0, pl.ds(g * L, L)] = mrl[pl.ds(c * CH + g * L, L)]
            ob[0, 0, pl.ds(g * L, L)] = mob[pl.ds(c * CH + g * L, L)]

    def _fire(slot):
        nr, rr, su, re_, _, sn, sr = slots[slot][:7]
        pltpu.async_copy(node_hbm.at[su.at[0, 0]], nr, sn)
        pltpu.async_copy(rel_hbm.at[re_.at[0, 0]], rr, sr)

    def _wait(slot):
        nr, rr, su, re_, _, sn, sr = slots[slot][:7]
        pltpu.make_async_copy(node_hbm.at[su.at[0, 0]], nr, sn).wait()
        pltpu.make_async_copy(rel_hbm.at[re_.at[0, 0]], rr, sr).wait()

    lane_dn = lax.GatherDimensionNumbers(
        offset_dims=(), collapsed_slice_dims=(0,), start_index_map=(0,))

    def _compute_scatter(c, slot):
        nr, rr, _, _, ob, _, _ = slots[slot][:7]
        ms = slots[slot][7]
        for g in range(GR):
            kv = mkg[pl.ds(c * CH + g * L, L)] * jnp.full((L,), D, jnp.int32)

            def _quad(q, carry):
                for u in range(4):
                    j2 = q * 4 + u
                    j = g * L + j2
                    perm = lax.broadcast(j2, (L,))
                    kgs = lax.gather(kv, perm[:, None], lane_dn,
                                     slice_sizes=(1,),
                                     mode=lax.GatherScatterMode.PROMISE_IN_BOUNDS)
                    acc = jnp.zeros((L,), jnp.float32)
                    for k in range(8):
                        a = nr[j, pl.ds(D + k * L, L)]
                        b = rr[j, pl.ds(D + k * L, L)]
                        cc = plsc.load_gather(c4v, [kgs + kconst[k]])
                        acc = acc + jnp.maximum(a + b + cc, 0.0) * wk[k]
                    sv = _lane_sum(acc) + b0
                    av = 1.0 / (1.0 + jnp.exp(-sv))
                    for k in range(8):
                        m = (nr[j, pl.ds(k * L, L)]
                             + rr[j, pl.ds(k * L, L)]) * av
                        ms[j, pl.ds(k * L, L)] = m
                return carry

            lax.fori_loop(0, 4, _quad, 0)
        pltpu.sync_copy(ms, acc_sh.at[ob.at[0, 0]], add=True)

    def _super(s_i, carry):
        sb = ebase + s_i * SUPER
        for col, buf in ((ekg_hbm, mkg), (erl_hbm, mrl), (esb_hbm, msb),
                         (eob_hbm, mob)):
            pltpu.sync_copy(col.at[pl.ds(sb, SUPER)], buf)

        _prep(0, 0)
        _fire(0)
        _prep(1, 1)
        _fire(1)

        def _pair(c2, carry2):
            c0 = c2 * 2
            _wait(0)
            _compute_scatter(c0, 0)
            _prep(c0 + 2, 0)
            _fire(0)
            _wait(1)
            _compute_scatter(c0 + 1, 1)
            _prep(c0 + 3, 1)
            _fire(1)
            return carry2

        lax.fori_loop(0, NP - 1, _pair, 0)
        _wait(0)
        _compute_scatter(NCH - 2, 0)
        _wait(1)
        _compute_scatter(NCH - 1, 1)
        return carry

    lax.fori_loop(0, NSUP, _super, 0)
    plsc.subcore_barrier()

    for i in range(NZ):
        r0 = sid * RW + i * ZR
        pltpu.sync_copy(acc_sh.at[pl.ds(r0, ZR)], zbuf)
        pltpu.sync_copy(zbuf, out_hbm.at[cid, pl.ds(r0, ZR)])


def _sc_edges(node_tab, rel_tab, c4, wvec, b0vec, ecols):
    ekg, erl, esb, eob = ecols
    mesh = plsc.VectorSubcoreMesh(core_axis_name="c", subcore_axis_name="s",
                                  num_cores=NC, num_subcores=NS)
    fn = pl.kernel(
        _sc_body,
        out_type=jax.ShapeDtypeStruct((NC, N_PAD, D), jnp.float32),
        mesh=mesh,
        compiler_params=pltpu.CompilerParams(needs_layout_passes=False),
        scratch_types=[
            pltpu.VMEM_SHARED((N_PAD, D), jnp.float32),
            pltpu.VMEM((CH, 2 * D), jnp.float32),
            pltpu.VMEM((CH, 2 * D), jnp.float32),
            pltpu.VMEM((CH, 2 * D), jnp.float32),
            pltpu.VMEM((CH, 2 * D), jnp.float32),
            pltpu.VMEM((CH, D), jnp.float32),
            pltpu.VMEM((ZR, D), jnp.float32),
            pltpu.VMEM((SUPER,), jnp.int32),
            pltpu.VMEM((SUPER,), jnp.int32),
            pltpu.VMEM((SUPER,), jnp.int32),
            pltpu.VMEM((SUPER,), jnp.int32),
            pltpu.VMEM((1, 1, CH), jnp.int32),
            pltpu.VMEM((1, 1, CH), jnp.int32),
            pltpu.VMEM((1, 1, CH), jnp.int32),
            pltpu.VMEM((1, 1, CH), jnp.int32),
            pltpu.VMEM((1, 1, CH), jnp.int32),
            pltpu.VMEM((1, 1, CH), jnp.int32),
            pltpu.VMEM((4 * D,), jnp.float32),
            pltpu.VMEM((D,), jnp.float32),
            pltpu.VMEM((L,), jnp.float32),
            pltpu.SemaphoreType.DMA,
            pltpu.SemaphoreType.DMA,
            pltpu.SemaphoreType.DMA,
            pltpu.SemaphoreType.DMA,
        ],
    )
    return fn(node_tab, rel_tab, c4, wvec, b0vec, ekg, erl, esb, eob)


def _fin_body(p_ref, wh_ref, out_ref):
    s = p_ref[0] + p_ref[1]
    out_ref[...] = lax.dot_general(
        s, wh_ref[...], (((1,), (1,)), ((), ())),
        preferred_element_type=jnp.float32)


def _finish(partials, wh):
    nblk = 8
    rows = N_PAD // nblk
    return pl.pallas_call(
        _fin_body,
        grid=(nblk,),
        in_specs=[
            pl.BlockSpec((NC, rows, D), lambda i: (0, i, 0)),
            pl.BlockSpec((D, D), lambda i: (0, 0)),
        ],
        out_specs=pl.BlockSpec((rows, D), lambda i: (i, 0)),
        out_shape=jax.ShapeDtypeStruct((N_PAD, D), jnp.float32),
    )(partials, wh)


def kernel(hidden, edges, n_node, kgemb, left_num, rela_embed, Ws, Wr,
           Wkg_W, Wkg_b, walpha_W, walpha_b, Wh):
    hid_p = jnp.pad(hidden, ((0, PAD_ROWS - hidden.shape[0]), (0, 0)))
    rel_p = jnp.pad(rela_embed, ((0, PAD_ROWS - rela_embed.shape[0]), (0, 0)))
    e32 = edges.astype(jnp.int32)
    npad = E_PAD - e32.shape[0]
    objc = jnp.minimum(e32[:, 5], hidden.shape[0] - 1)
    objc = jnp.pad(objc, (0, npad), constant_values=N_NODE)
    kidx = 2 * (e32[:, 1] >= left_num).astype(jnp.int32) \
        + (e32[:, 3] >= left_num).astype(jnp.int32)
    ecols = (jnp.pad(kidx, (0, npad)), jnp.pad(e32[:, 2], (0, npad)),
             jnp.pad(e32[:, 4], (0, npad)), objc)
    wvec = walpha_W.reshape(D)
    b0vec = jnp.broadcast_to(walpha_b.reshape(1), (L,)).astype(jnp.float32)

    node_tab, rel_tab = _build_tables(hid_p, rel_p, Ws, Wr)
    c4 = _build_c4(kgemb, Wkg_W, Wkg_b)[:4].reshape(4 * D)
    partials = _sc_edges(node_tab, rel_tab, c4, wvec, b0vec, ecols)
    return _finish(partials, Wh)[:N_NODE]


# speedup vs baseline: 1.3234x; 1.0805x over previous
import functools

import numpy as np

import jax
import jax.numpy as jnp
from jax import lax
from jax.experimental import pallas as pl
from jax.experimental.pallas import tpu as pltpu
from jax.experimental.pallas import tpu_sc as plsc

N_NODE = 10000
D = 128
L = 16
NC, NS = 2, 16
NW = NC * NS
EW = 10240
E_PAD = NW * EW
SUPER = 1280
NSUP = EW // SUPER
CH = 32
NCH = SUPER // CH
NP = NCH // 2
GR = CH // L
N_PAD = 10240
RW = N_PAD // NS
ZR = 16
NZ = RW // ZR
PAD_ROWS = 10048
SIGMA = np.concatenate([np.concatenate([np.arange(32 * m, 32 * m + 32, 2),
                                        np.arange(32 * m + 1, 32 * m + 32, 2)])
                        for m in range(4)])


def _tables_body(hid_ref, rel_ref, ws_ref, wr_ref, node_ref, relo_ref):
    h = hid_ref[...]
    r = rel_ref[...]
    node_ref[:, :D] = h.astype(jnp.bfloat16)
    node_ref[:, D:] = lax.dot_general(
        h, ws_ref[...], (((1,), (1,)), ((), ())),
        preferred_element_type=jnp.float32).astype(jnp.bfloat16)
    relo_ref[:, :D] = r.astype(jnp.bfloat16)
    relo_ref[:, D:] = lax.dot_general(
        r, wr_ref[...], (((1,), (1,)), ((), ())),
        preferred_element_type=jnp.float32).astype(jnp.bfloat16)


def _build_tables(hid_p, rel_p, ws, wr):
    nblk = 8
    rows = PAD_ROWS // nblk
    return pl.pallas_call(
        _tables_body,
        grid=(nblk,),
        in_specs=[
            pl.BlockSpec((rows, D), lambda i: (i, 0)),
            pl.BlockSpec((rows, D), lambda i: (i, 0)),
            pl.BlockSpec((D, D), lambda i: (0, 0)),
            pl.BlockSpec((D, D), lambda i: (0, 0)),
        ],
        out_specs=[
            pl.BlockSpec((rows, 2 * D), lambda i: (i, 0)),
            pl.BlockSpec((rows, 2 * D), lambda i: (i, 0)),
        ],
        out_shape=[
            jax.ShapeDtypeStruct((PAD_ROWS, 2 * D), jnp.bfloat16),
            jax.ShapeDtypeStruct((PAD_ROWS, 2 * D), jnp.bfloat16),
        ],
    )(hid_p, rel_p, ws, wr)


def _c4_body(kg_ref, wkg_ref, wb_ref, out_ref):
    kg = kg_ref[...]
    w1 = wkg_ref[:, :D]
    w2 = wkg_ref[:, D:]
    kg1 = lax.dot_general(kg, w1, (((1,), (1,)), ((), ())),
                          preferred_element_type=jnp.float32)
    kg2 = lax.dot_general(kg, w2, (((1,), (1,)), ((), ())),
                          preferred_element_type=jnp.float32)
    c = kg1[:, None, :] + kg2[None, :, :] + wb_ref[...][None, None, :]
    c = c.reshape(4, D)
    out_ref[...] = jnp.concatenate([c, jnp.zeros((4, D), jnp.float32)], axis=0)


def _build_c4(kgemb, wkg_w, wkg_b):
    return pl.pallas_call(
        _c4_body,
        out_shape=jax.ShapeDtypeStruct((8, D), jnp.float32),
    )(kgemb, wkg_w, wkg_b)


def _lane_sum(v):
    lanes = lax.iota(jnp.int32, L)
    dnums = lax.GatherDimensionNumbers(
        offset_dims=(), collapsed_slice_dims=(0,), start_index_map=(0,))
    for sh in (1, 2, 4, 8):
        perm = lax.bitwise_xor(lanes, jnp.full((L,), sh, jnp.int32))
        v = v + lax.gather(v, perm[:, None], dnums, slice_sizes=(1,),
                           mode=lax.GatherScatterMode.PROMISE_IN_BOUNDS)
    return v


def _sc_body(node_hbm, rel_hbm, c4_hbm, w_hbm, b0_hbm, ekg_hbm,
             erl_hbm, esb_hbm, eob_hbm,
             out_hbm, acc_sh, nrows0, nrows1, rrows0, rrows1, msg0,
             zbuf, mkg, mrl, msb, mob, subi0, subi1, reli0, reli1, obji0,
             obji1, c4v, wv, b0v, semn0, semr0, semn1, semr1):
    cid = lax.axis_index("c")
    sid = lax.axis_index("s")
    wid = cid * NS + sid

    pltpu.sync_copy(c4_hbm, c4v)
    pltpu.sync_copy(w_hbm, wv)
    pltpu.sync_copy(b0_hbm, b0v)

    zero = jnp.zeros((L,), jnp.float32)

    def _zrow(r, carry):
        for k in range(8):
            zbuf[r, pl.ds(k * L, L)] = zero
        return carry

    lax.fori_loop(0, ZR, _zrow, 0)
    for i in range(NZ):
        pltpu.sync_copy(zbuf, acc_sh.at[pl.ds(sid * RW + i * ZR, ZR)])
    plsc.subcore_barrier()

    wk = tuple(wv[pl.ds(k * L, L)] for k in range(8))
    b0 = b0v[...]
    kconst = tuple(lax.iota(jnp.int32, L) + jnp.full((L,), k * L, jnp.int32)
                   for k in range(8))

    ebase = wid * EW
    slots = ((nrows0, rrows0, subi0, reli0, obji0, semn0, semr0, msg0),
             (nrows1, rrows1, subi1, reli1, obji1, semn1, semr1, msg0))

    def _prep(c, slot):
        su, re_, ob = slots[slot][2:5]
        for g in range(GR):
            su[0, 0, pl.ds(g * L, L)] = msb[pl.ds(c * CH + g * L, L)]
            re_[0, 0, pl.ds(g * L, L)] = mrl[pl.ds(c * CH + g * L, L)]
            ob[0, 0, pl.ds(g * L, L)] = mob[pl.ds(c * CH + g * L, L)]

    def _fire(slot):
        nr, rr, su, re_, _, sn, sr = slots[slot][:7]
        pltpu.async_copy(node_hbm.at[su.at[0, 0]], nr, sn)
        pltpu.async_copy(rel_hbm.at[re_.at[0, 0]], rr, sr)

    def _wait(slot):
        nr, rr, su, re_, _, sn, sr = slots[slot][:7]
        pltpu.make_async_copy(node_hbm.at[su.at[0, 0]], nr, sn).wait()
        pltpu.make_async_copy(rel_hbm.at[re_.at[0, 0]], rr, sr).wait()

    lane_dn = lax.GatherDimensionNumbers(
        offset_dims=(), collapsed_slice_dims=(0,), start_index_map=(0,))

    def _compute_scatter(c, slot):
        nr, rr, _, _, ob, _, _ = slots[slot][:7]
        ms = slots[slot][7]
        for g in range(GR):
            kv = mkg[pl.ds(c * CH + g * L, L)] * jnp.full((L,), D, jnp.int32)

            def _quad(q, carry):
                for u in range(4):
                    j2 = q * 4 + u
                    j = g * L + j2
                    perm = lax.broadcast(j2, (L,))
                    kgs = lax.gather(kv, perm[:, None], lane_dn,
                                     slice_sizes=(1,),
                                     mode=lax.GatherScatterMode.PROMISE_IN_BOUNDS)
                    acc = jnp.zeros((L,), jnp.float32)
                    for m in range(4):
                        a_ev, a_od = plsc.unpack(
                            plsc.bitcast(nr[j, pl.ds(4 * L + m * L, L)],
                                         jnp.bfloat16),
                            format=plsc.PackFormat.INTERLEAVED)
                        b_ev, b_od = plsc.unpack(
                            plsc.bitcast(rr[j, pl.ds(4 * L + m * L, L)],
                                         jnp.bfloat16),
                            format=plsc.PackFormat.INTERLEAVED)
                        c_ev = plsc.load_gather(c4v, [kgs + kconst[2 * m]])
                        c_od = plsc.load_gather(c4v, [kgs + kconst[2 * m + 1]])
                        acc = (acc
                               + jnp.maximum(a_ev + b_ev + c_ev, 0.0) * wk[2 * m]
                               + jnp.maximum(a_od + b_od + c_od, 0.0) * wk[2 * m + 1])
                    sv = _lane_sum(acc) + b0
                    av = 1.0 / (1.0 + jnp.exp(-sv))
                    for m in range(4):
                        h_ev, h_od = plsc.unpack(
                            plsc.bitcast(nr[j, pl.ds(m * L, L)],
                                         jnp.bfloat16),
                            format=plsc.PackFormat.INTERLEAVED)
                        r_ev, r_od = plsc.unpack(
                            plsc.bitcast(rr[j, pl.ds(m * L, L)],
                                         jnp.bfloat16),
                            format=plsc.PackFormat.INTERLEAVED)
                        ms[j, pl.ds(m * 2 * L, L)] = (h_ev + r_ev) * av
                        ms[j, pl.ds(m * 2 * L + L, L)] = (h_od + r_od) * av
                return carry

            lax.fori_loop(0, 4, _quad, 0)
        pltpu.sync_copy(ms, acc_sh.at[ob.at[0, 0]], add=True)

    def _super(s_i, carry):
        sb = ebase + s_i * SUPER
        for col, buf in ((ekg_hbm, mkg), (erl_hbm, mrl), (esb_hbm, msb),
                         (eob_hbm, mob)):
            pltpu.sync_copy(col.at[pl.ds(sb, SUPER)], buf)

        _prep(0, 0)
        _fire(0)
        _prep(1, 1)
        _fire(1)

        def _pair(c2, carry2):
            c0 = c2 * 2
            _wait(0)
            _compute_scatter(c0, 0)
            _prep(c0 + 2, 0)
            _fire(0)
            _wait(1)
            _compute_scatter(c0 + 1, 1)
            _prep(c0 + 3, 1)
            _fire(1)
            return carry2

        lax.fori_loop(0, NP - 1, _pair, 0)
        _wait(0)
        _compute_scatter(NCH - 2, 0)
        _wait(1)
        _compute_scatter(NCH - 1, 1)
        return carry

    lax.fori_loop(0, NSUP, _super, 0)
    plsc.subcore_barrier()

    for i in range(NZ):
        r0 = sid * RW + i * ZR
        pltpu.sync_copy(acc_sh.at[pl.ds(r0, ZR)], zbuf)
        pltpu.sync_copy(zbuf, out_hbm.at[cid, pl.ds(r0, ZR)])


def _sc_edges(node_tab, rel_tab, c4, wvec, b0vec, ecols):
    ekg, erl, esb, eob = ecols
    mesh = plsc.VectorSubcoreMesh(core_axis_name="c", subcore_axis_name="s",
                                  num_cores=NC, num_subcores=NS)
    fn = pl.kernel(
        _sc_body,
        out_type=jax.ShapeDtypeStruct((NC, N_PAD, D), jnp.float32),
        mesh=mesh,
        compiler_params=pltpu.CompilerParams(needs_layout_passes=False),
        scratch_types=[
            pltpu.VMEM_SHARED((N_PAD, D), jnp.float32),
            pltpu.VMEM((CH, D), jnp.int32),
            pltpu.VMEM((CH, D), jnp.int32),
            pltpu.VMEM((CH, D), jnp.int32),
            pltpu.VMEM((CH, D), jnp.int32),
            pltpu.VMEM((CH, D), jnp.float32),
            pltpu.VMEM((ZR, D), jnp.float32),
            pltpu.VMEM((SUPER,), jnp.int32),
            pltpu.VMEM((SUPER,), jnp.int32),
            pltpu.VMEM((SUPER,), jnp.int32),
            pltpu.VMEM((SUPER,), jnp.int32),
            pltpu.VMEM((1, 1, CH), jnp.int32),
            pltpu.VMEM((1, 1, CH), jnp.int32),
            pltpu.VMEM((1, 1, CH), jnp.int32),
            pltpu.VMEM((1, 1, CH), jnp.int32),
            pltpu.VMEM((1, 1, CH), jnp.int32),
            pltpu.VMEM((1, 1, CH), jnp.int32),
            pltpu.VMEM((4 * D,), jnp.float32),
            pltpu.VMEM((D,), jnp.float32),
            pltpu.VMEM((L,), jnp.float32),
            pltpu.SemaphoreType.DMA,
            pltpu.SemaphoreType.DMA,
            pltpu.SemaphoreType.DMA,
            pltpu.SemaphoreType.DMA,
        ],
    )
    return fn(node_tab, rel_tab, c4, wvec, b0vec, ekg, erl, esb, eob)


def _fin_body(p_ref, wh_ref, out_ref):
    s = p_ref[0] + p_ref[1]
    out_ref[...] = lax.dot_general(
        s, wh_ref[...], (((1,), (1,)), ((), ())),
        preferred_element_type=jnp.float32)


def _finish(partials, wh):
    nblk = 8
    rows = N_PAD // nblk
    return pl.pallas_call(
        _fin_body,
        grid=(nblk,),
        in_specs=[
            pl.BlockSpec((NC, rows, D), lambda i: (0, i, 0)),
            pl.BlockSpec((D, D), lambda i: (0, 0)),
        ],
        out_specs=pl.BlockSpec((rows, D), lambda i: (i, 0)),
        out_shape=jax.ShapeDtypeStruct((N_PAD, D), jnp.float32),
    )(partials, wh)


def kernel(hidden, edges, n_node, kgemb, left_num, rela_embed, Ws, Wr,
           Wkg_W, Wkg_b, walpha_W, walpha_b, Wh):
    hid_p = jnp.pad(hidden, ((0, PAD_ROWS - hidden.shape[0]), (0, 0)))
    rel_p = jnp.pad(rela_embed, ((0, PAD_ROWS - rela_embed.shape[0]), (0, 0)))
    e32 = edges.astype(jnp.int32)
    npad = E_PAD - e32.shape[0]
    objc = jnp.minimum(e32[:, 5], hidden.shape[0] - 1)
    objc = jnp.pad(objc, (0, npad), constant_values=N_NODE)
    kidx = 2 * (e32[:, 1] >= left_num).astype(jnp.int32) \
        + (e32[:, 3] >= left_num).astype(jnp.int32)
    ecols = (jnp.pad(kidx, (0, npad)), jnp.pad(e32[:, 2], (0, npad)),
             jnp.pad(e32[:, 4], (0, npad)), objc)
    wvec = walpha_W.reshape(D)[SIGMA]
    b0vec = jnp.broadcast_to(walpha_b.reshape(1), (L,)).astype(jnp.float32)

    node_tab, rel_tab = _build_tables(hid_p, rel_p, Ws, Wr)
    node_tab = lax.bitcast_convert_type(
        node_tab.reshape(PAD_ROWS, D, 2), jnp.int32)
    rel_tab = lax.bitcast_convert_type(
        rel_tab.reshape(PAD_ROWS, D, 2), jnp.int32)
    c4 = _build_c4(kgemb, Wkg_W, Wkg_b)[:4][:, SIGMA].reshape(4 * D)
    partials = _sc_edges(node_tab, rel_tab, c4, wvec, b0vec, ecols)
    return _finish(partials, Wh[:, SIGMA])[:N_NODE]


# speedup vs baseline: 1.3264x; 1.0022x over previous
import functools

import numpy as np

import jax
import jax.numpy as jnp
from jax import lax
from jax.experimental import pallas as pl
from jax.experimental.pallas import tpu as pltpu
from jax.experimental.pallas import tpu_sc as plsc

N_NODE = 10000
D = 128
L = 16
NC, NS = 2, 16
NW = NC * NS
EW = 10240
E_PAD = NW * EW
SUPER = 1280
NSUP = EW // SUPER
CH = 32
NCH = SUPER // CH
NP = NCH // 2
GR = CH // L
N_PAD = 10240
RW = N_PAD // NS
ZR = 16
NZ = RW // ZR
PAD_ROWS = 10048
SIGMA = np.concatenate([np.concatenate([np.arange(32 * m, 32 * m + 32, 2),
                                        np.arange(32 * m + 1, 32 * m + 32, 2)])
                        for m in range(4)])


def _tables_body(hid_ref, rel_ref, ws_ref, wr_ref, node_ref, relo_ref):
    h = hid_ref[...]
    r = rel_ref[...]
    node_ref[:, :D] = h.astype(jnp.bfloat16)
    node_ref[:, D:] = lax.dot_general(
        h, ws_ref[...], (((1,), (1,)), ((), ())),
        preferred_element_type=jnp.float32).astype(jnp.bfloat16)
    relo_ref[:, :D] = r.astype(jnp.bfloat16)
    relo_ref[:, D:] = lax.dot_general(
        r, wr_ref[...], (((1,), (1,)), ((), ())),
        preferred_element_type=jnp.float32).astype(jnp.bfloat16)


def _build_tables(hid_p, rel_p, ws, wr):
    nblk = 8
    rows = PAD_ROWS // nblk
    return pl.pallas_call(
        _tables_body,
        grid=(nblk,),
        in_specs=[
            pl.BlockSpec((rows, D), lambda i: (i, 0)),
            pl.BlockSpec((rows, D), lambda i: (i, 0)),
            pl.BlockSpec((D, D), lambda i: (0, 0)),
            pl.BlockSpec((D, D), lambda i: (0, 0)),
        ],
        out_specs=[
            pl.BlockSpec((rows, 2 * D), lambda i: (i, 0)),
            pl.BlockSpec((rows, 2 * D), lambda i: (i, 0)),
        ],
        out_shape=[
            jax.ShapeDtypeStruct((PAD_ROWS, 2 * D), jnp.bfloat16),
            jax.ShapeDtypeStruct((PAD_ROWS, 2 * D), jnp.bfloat16),
        ],
    )(hid_p, rel_p, ws, wr)


def _c4_body(kg_ref, wkg_ref, wb_ref, out_ref):
    kg = kg_ref[...]
    w1 = wkg_ref[:, :D]
    w2 = wkg_ref[:, D:]
    kg1 = lax.dot_general(kg, w1, (((1,), (1,)), ((), ())),
                          preferred_element_type=jnp.float32)
    kg2 = lax.dot_general(kg, w2, (((1,), (1,)), ((), ())),
                          preferred_element_type=jnp.float32)
    c = kg1[:, None, :] + kg2[None, :, :] + wb_ref[...][None, None, :]
    c = c.reshape(4, D)
    out_ref[...] = jnp.concatenate([c, jnp.zeros((4, D), jnp.float32)], axis=0)


def _build_c4(kgemb, wkg_w, wkg_b):
    return pl.pallas_call(
        _c4_body,
        out_shape=jax.ShapeDtypeStruct((8, D), jnp.float32),
    )(kgemb, wkg_w, wkg_b)


def _lane_sum(v):
    lanes = lax.iota(jnp.int32, L)
    dnums = lax.GatherDimensionNumbers(
        offset_dims=(), collapsed_slice_dims=(0,), start_index_map=(0,))
    for sh in (1, 2, 4, 8):
        perm = lax.bitwise_xor(lanes, jnp.full((L,), sh, jnp.int32))
        v = v + lax.gather(v, perm[:, None], dnums, slice_sizes=(1,),
                           mode=lax.GatherScatterMode.PROMISE_IN_BOUNDS)
    return v


def _sc_body(node_hbm, rel_hbm, c4_hbm, w_hbm, b0_hbm, ekg_hbm,
             erl_hbm, esb_hbm, eob_hbm,
             out_hbm, acc_sh, nrows0, nrows1, rrows0, rrows1, msg0,
             zbuf, mkg, mrl, msb, mob, subi0, subi1, reli0, reli1, obji0,
             obji1, c4v, wv, b0v, semn0, semr0, semn1, semr1):
    cid = lax.axis_index("c")
    sid = lax.axis_index("s")
    wid = cid * NS + sid

    pltpu.sync_copy(c4_hbm, c4v)
    pltpu.sync_copy(w_hbm, wv)
    pltpu.sync_copy(b0_hbm, b0v)

    zero = jnp.zeros((L,), jnp.float32)

    def _zrow(r, carry):
        for k in range(8):
            zbuf[r, pl.ds(k * L, L)] = zero
        return carry

    lax.fori_loop(0, ZR, _zrow, 0)
    for i in range(NZ):
        pltpu.sync_copy(zbuf, acc_sh.at[pl.ds(sid * RW + i * ZR, ZR)])
    plsc.subcore_barrier()

    wk = tuple(wv[pl.ds(k * L, L)] for k in range(8))
    b0 = b0v[...]
    kconst = tuple(lax.iota(jnp.int32, L) + jnp.full((L,), k * L, jnp.int32)
                   for k in range(8))

    ebase = wid * EW
    slots = ((nrows0, rrows0, subi0, reli0, obji0, semn0, semr0, msg0),
             (nrows1, rrows1, subi1, reli1, obji1, semn1, semr1, msg0))

    def _prep(c, slot):
        su, re_, ob = slots[slot][2:5]
        for g in range(GR):
            su[0, 0, pl.ds(g * L, L)] = msb[pl.ds(c * CH + g * L, L)]
            re_[0, 0, pl.ds(g * L, L)] = mrl[pl.ds(c * CH + g * L, L)]
            ob[0, 0, pl.ds(g * L, L)] = mob[pl.ds(c * CH + g * L, L)]

    def _fire(slot):
        nr, rr, su, re_, _, sn, sr = slots[slot][:7]
        pltpu.async_copy(node_hbm.at[su.at[0, 0]], nr, sn)
        pltpu.async_copy(rel_hbm.at[re_.at[0, 0]], rr, sr)

    def _wait(slot):
        nr, rr, su, re_, _, sn, sr = slots[slot][:7]
        pltpu.make_async_copy(node_hbm.at[su.at[0, 0]], nr, sn).wait()
        pltpu.make_async_copy(rel_hbm.at[re_.at[0, 0]], rr, sr).wait()

    lane_dn = lax.GatherDimensionNumbers(
        offset_dims=(), collapsed_slice_dims=(0,), start_index_map=(0,))

    def _compute_scatter(c, slot):
        nr, rr, _, _, ob, _, _ = slots[slot][:7]
        ms = slots[slot][7]
        for g in range(GR):
            kv = mkg[pl.ds(c * CH + g * L, L)] * jnp.full((L,), D, jnp.int32)

            def _quad(q, carry):
                for u in range(4):
                    j2 = q * 4 + u
                    j = g * L + j2
                    perm = lax.broadcast(j2, (L,))
                    kgs = lax.gather(kv, perm[:, None], lane_dn,
                                     slice_sizes=(1,),
                                     mode=lax.GatherScatterMode.PROMISE_IN_BOUNDS)
                    acc_e = jnp.zeros((L,), jnp.float32)
                    acc_o = jnp.zeros((L,), jnp.float32)
                    for m in range(4):
                        a_ev, a_od = plsc.unpack(
                            plsc.bitcast(nr[j, pl.ds(4 * L + m * L, L)],
                                         jnp.bfloat16),
                            format=plsc.PackFormat.INTERLEAVED)
                        b_ev, b_od = plsc.unpack(
                            plsc.bitcast(rr[j, pl.ds(4 * L + m * L, L)],
                                         jnp.bfloat16),
                            format=plsc.PackFormat.INTERLEAVED)
                        c_ev = plsc.load_gather(c4v, [kgs + kconst[2 * m]])
                        c_od = plsc.load_gather(c4v, [kgs + kconst[2 * m + 1]])
                        acc_e = acc_e + jnp.maximum(a_ev + b_ev + c_ev, 0.0) * wk[2 * m]
                        acc_o = acc_o + jnp.maximum(a_od + b_od + c_od, 0.0) * wk[2 * m + 1]
                    sv = _lane_sum(acc_e + acc_o) + b0
                    av = 1.0 / (1.0 + jnp.exp(-sv))
                    for m in range(4):
                        h_ev, h_od = plsc.unpack(
                            plsc.bitcast(nr[j, pl.ds(m * L, L)],
                                         jnp.bfloat16),
                            format=plsc.PackFormat.INTERLEAVED)
                        r_ev, r_od = plsc.unpack(
                            plsc.bitcast(rr[j, pl.ds(m * L, L)],
                                         jnp.bfloat16),
                            format=plsc.PackFormat.INTERLEAVED)
                        ms[j, pl.ds(m * 2 * L, L)] = (h_ev + r_ev) * av
                        ms[j, pl.ds(m * 2 * L + L, L)] = (h_od + r_od) * av
                return carry

            lax.fori_loop(0, 4, _quad, 0)
        pltpu.sync_copy(ms, acc_sh.at[ob.at[0, 0]], add=True)

    def _super(s_i, carry):
        sb = ebase + s_i * SUPER
        for col, buf in ((ekg_hbm, mkg), (erl_hbm, mrl), (esb_hbm, msb),
                         (eob_hbm, mob)):
            pltpu.sync_copy(col.at[pl.ds(sb, SUPER)], buf)

        _prep(0, 0)
        _fire(0)
        _prep(1, 1)
        _fire(1)

        def _pair(c2, carry2):
            c0 = c2 * 2
            _wait(0)
            _compute_scatter(c0, 0)
            _prep(c0 + 2, 0)
            _fire(0)
            _wait(1)
            _compute_scatter(c0 + 1, 1)
            _prep(c0 + 3, 1)
            _fire(1)
            return carry2

        lax.fori_loop(0, NP - 1, _pair, 0)
        _wait(0)
        _compute_scatter(NCH - 2, 0)
        _wait(1)
        _compute_scatter(NCH - 1, 1)
        return carry

    lax.fori_loop(0, NSUP, _super, 0)
    plsc.subcore_barrier()

    for i in range(NZ):
        r0 = sid * RW + i * ZR
        pltpu.sync_copy(acc_sh.at[pl.ds(r0, ZR)], zbuf)
        pltpu.sync_copy(zbuf, out_hbm.at[cid, pl.ds(r0, ZR)])


def _sc_edges(node_tab, rel_tab, c4, wvec, b0vec, ecols):
    ekg, erl, esb, eob = ecols
    mesh = plsc.VectorSubcoreMesh(core_axis_name="c", subcore_axis_name="s",
                                  num_cores=NC, num_subcores=NS)
    fn = pl.kernel(
        _sc_body,
        out_type=jax.ShapeDtypeStruct((NC, N_PAD, D), jnp.float32),
        mesh=mesh,
        compiler_params=pltpu.CompilerParams(needs_layout_passes=False),
        scratch_types=[
            pltpu.VMEM_SHARED((N_PAD, D), jnp.float32),
            pltpu.VMEM((CH, D), jnp.int32),
            pltpu.VMEM((CH, D), jnp.int32),
            pltpu.VMEM((CH, D), jnp.int32),
            pltpu.VMEM((CH, D), jnp.int32),
            pltpu.VMEM((CH, D), jnp.float32),
            pltpu.VMEM((ZR, D), jnp.float32),
            pltpu.VMEM((SUPER,), jnp.int32),
            pltpu.VMEM((SUPER,), jnp.int32),
            pltpu.VMEM((SUPER,), jnp.int32),
            pltpu.VMEM((SUPER,), jnp.int32),
            pltpu.VMEM((1, 1, CH), jnp.int32),
            pltpu.VMEM((1, 1, CH), jnp.int32),
            pltpu.VMEM((1, 1, CH), jnp.int32),
            pltpu.VMEM((1, 1, CH), jnp.int32),
            pltpu.VMEM((1, 1, CH), jnp.int32),
            pltpu.VMEM((1, 1, CH), jnp.int32),
            pltpu.VMEM((4 * D,), jnp.float32),
            pltpu.VMEM((D,), jnp.float32),
            pltpu.VMEM((L,), jnp.float32),
            pltpu.SemaphoreType.DMA,
            pltpu.SemaphoreType.DMA,
            pltpu.SemaphoreType.DMA,
            pltpu.SemaphoreType.DMA,
        ],
    )
    return fn(node_tab, rel_tab, c4, wvec, b0vec, ekg, erl, esb, eob)


def _fin_body(p_ref, wh_ref, out_ref):
    s = p_ref[0] + p_ref[1]
    out_ref[...] = lax.dot_general(
        s, wh_ref[...], (((1,), (1,)), ((), ())),
        preferred_element_type=jnp.float32)


def _finish(partials, wh):
    nblk = 8
    rows = N_PAD // nblk
    return pl.pallas_call(
        _fin_body,
        grid=(nblk,),
        in_specs=[
            pl.BlockSpec((NC, rows, D), lambda i: (0, i, 0)),
            pl.BlockSpec((D, D), lambda i: (0, 0)),
        ],
        out_specs=pl.BlockSpec((rows, D), lambda i: (i, 0)),
        out_shape=jax.ShapeDtypeStruct((N_PAD, D), jnp.float32),
    )(partials, wh)


def kernel(hidden, edges, n_node, kgemb, left_num, rela_embed, Ws, Wr,
           Wkg_W, Wkg_b, walpha_W, walpha_b, Wh):
    hid_p = jnp.pad(hidden, ((0, PAD_ROWS - hidden.shape[0]), (0, 0)))
    rel_p = jnp.pad(rela_embed, ((0, PAD_ROWS - rela_embed.shape[0]), (0, 0)))
    e32 = edges.astype(jnp.int32)
    npad = E_PAD - e32.shape[0]
    objc = jnp.minimum(e32[:, 5], hidden.shape[0] - 1)
    objc = jnp.pad(objc, (0, npad), constant_values=N_NODE)
    kidx = 2 * (e32[:, 1] >= left_num).astype(jnp.int32) \
        + (e32[:, 3] >= left_num).astype(jnp.int32)
    ecols = (jnp.pad(kidx, (0, npad)), jnp.pad(e32[:, 2], (0, npad)),
             jnp.pad(e32[:, 4], (0, npad)), objc)
    wvec = walpha_W.reshape(D)[SIGMA]
    b0vec = jnp.broadcast_to(walpha_b.reshape(1), (L,)).astype(jnp.float32)

    node_tab, rel_tab = _build_tables(hid_p, rel_p, Ws, Wr)
    node_tab = lax.bitcast_convert_type(
        node_tab.reshape(PAD_ROWS, D, 2), jnp.int32)
    rel_tab = lax.bitcast_convert_type(
        rel_tab.reshape(PAD_ROWS, D, 2), jnp.int32)
    c4 = _build_c4(kgemb, Wkg_W, Wkg_b)[:4][:, SIGMA].reshape(4 * D)
    partials = _sc_edges(node_tab, rel_tab, c4, wvec, b0vec, ecols)
    return _finish(partials, Wh[:, SIGMA])[:N_NODE]


# speedup vs baseline: 1.4005x; 1.0559x over previous
import functools

import numpy as np

import jax
import jax.numpy as jnp
from jax import lax
from jax.experimental import pallas as pl
from jax.experimental.pallas import tpu as pltpu
from jax.experimental.pallas import tpu_sc as plsc

N_NODE = 10000
D = 128
L = 16
NC, NS = 2, 16
NW = NC * NS
EW = 10240
E_PAD = NW * EW
SUPER = 1280
NSUP = EW // SUPER
CH = 32
NCH = SUPER // CH
NP = NCH // 2
GR = CH // L
N_PAD = 10240
RW = N_PAD // NS
ZR = 16
NZ = RW // ZR
PAD_ROWS = 10048
SIGMA = np.concatenate([np.concatenate([np.arange(32 * m, 32 * m + 32, 2),
                                        np.arange(32 * m + 1, 32 * m + 32, 2)])
                        for m in range(4)])


def _tables_body(hid_ref, rel_ref, ws_ref, wr_ref, node_ref, relo_ref):
    h = hid_ref[...]
    r = rel_ref[...]
    node_ref[:, :D] = h.astype(jnp.bfloat16)
    node_ref[:, D:] = lax.dot_general(
        h, ws_ref[...], (((1,), (1,)), ((), ())),
        preferred_element_type=jnp.float32).astype(jnp.bfloat16)
    relo_ref[:, :D] = r.astype(jnp.bfloat16)
    relo_ref[:, D:] = lax.dot_general(
        r, wr_ref[...], (((1,), (1,)), ((), ())),
        preferred_element_type=jnp.float32).astype(jnp.bfloat16)


def _build_tables(hid_p, rel_p, ws, wr):
    nblk = 8
    rows = PAD_ROWS // nblk
    return pl.pallas_call(
        _tables_body,
        grid=(nblk,),
        in_specs=[
            pl.BlockSpec((rows, D), lambda i: (i, 0)),
            pl.BlockSpec((rows, D), lambda i: (i, 0)),
            pl.BlockSpec((D, D), lambda i: (0, 0)),
            pl.BlockSpec((D, D), lambda i: (0, 0)),
        ],
        out_specs=[
            pl.BlockSpec((rows, 2 * D), lambda i: (i, 0)),
            pl.BlockSpec((rows, 2 * D), lambda i: (i, 0)),
        ],
        out_shape=[
            jax.ShapeDtypeStruct((PAD_ROWS, 2 * D), jnp.bfloat16),
            jax.ShapeDtypeStruct((PAD_ROWS, 2 * D), jnp.bfloat16),
        ],
    )(hid_p, rel_p, ws, wr)


def _c4_body(kg_ref, wkg_ref, wb_ref, out_ref):
    kg = kg_ref[...]
    w1 = wkg_ref[:, :D]
    w2 = wkg_ref[:, D:]
    kg1 = lax.dot_general(kg, w1, (((1,), (1,)), ((), ())),
                          preferred_element_type=jnp.float32)
    kg2 = lax.dot_general(kg, w2, (((1,), (1,)), ((), ())),
                          preferred_element_type=jnp.float32)
    c = kg1[:, None, :] + kg2[None, :, :] + wb_ref[...][None, None, :]
    c = c.reshape(4, D)
    out_ref[...] = jnp.concatenate([c, jnp.zeros((4, D), jnp.float32)], axis=0)


def _build_c4(kgemb, wkg_w, wkg_b):
    return pl.pallas_call(
        _c4_body,
        out_shape=jax.ShapeDtypeStruct((8, D), jnp.float32),
    )(kgemb, wkg_w, wkg_b)


def _lane_sum(v):
    lanes = lax.iota(jnp.int32, L)
    dnums = lax.GatherDimensionNumbers(
        offset_dims=(), collapsed_slice_dims=(0,), start_index_map=(0,))
    for sh in (1, 2, 4, 8):
        perm = lax.bitwise_xor(lanes, jnp.full((L,), sh, jnp.int32))
        v = v + lax.gather(v, perm[:, None], dnums, slice_sizes=(1,),
                           mode=lax.GatherScatterMode.PROMISE_IN_BOUNDS)
    return v


def _sc_body(node_hbm, rel_hbm, c4_hbm, w_hbm, b0_hbm, ekg_hbm,
             erl_hbm, esb_hbm, eob_hbm,
             out_hbm, acc_sh, nrows0, nrows1, rrows0, rrows1, msg0, msg1,
             zbuf, mkg, mrl, msb, mob, subi0, subi1, reli0, reli1, obji0,
             obji1, obsc0, obsc1, c4v, wv, b0v, semn0, semr0, semn1, semr1,
             sems0, sems1):
    cid = lax.axis_index("c")
    sid = lax.axis_index("s")
    wid = cid * NS + sid

    pltpu.sync_copy(c4_hbm, c4v)
    pltpu.sync_copy(w_hbm, wv)
    pltpu.sync_copy(b0_hbm, b0v)

    zero = jnp.zeros((L,), jnp.float32)

    def _zrow(r, carry):
        for k in range(8):
            zbuf[r, pl.ds(k * L, L)] = zero
        return carry

    lax.fori_loop(0, ZR, _zrow, 0)
    for i in range(NZ):
        pltpu.sync_copy(zbuf, acc_sh.at[pl.ds(sid * RW + i * ZR, ZR)])
    plsc.subcore_barrier()

    wk = tuple(wv[pl.ds(k * L, L)] for k in range(8))
    b0 = b0v[...]
    kconst = tuple(lax.iota(jnp.int32, L) + jnp.full((L,), k * L, jnp.int32)
                   for k in range(8))

    ebase = wid * EW
    slots = ((nrows0, rrows0, subi0, reli0, obji0, semn0, semr0, msg0,
              sems0, obsc0),
             (nrows1, rrows1, subi1, reli1, obji1, semn1, semr1, msg1,
              sems1, obsc1))

    def _prep(c, slot):
        su, re_, ob = slots[slot][2:5]
        for g in range(GR):
            su[0, 0, pl.ds(g * L, L)] = msb[pl.ds(c * CH + g * L, L)]
            re_[0, 0, pl.ds(g * L, L)] = mrl[pl.ds(c * CH + g * L, L)]
            ob[0, 0, pl.ds(g * L, L)] = mob[pl.ds(c * CH + g * L, L)]

    def _fire(slot):
        nr, rr, su, re_, _, sn, sr = slots[slot][:7]
        pltpu.async_copy(node_hbm.at[su.at[0, 0]], nr, sn)
        pltpu.async_copy(rel_hbm.at[re_.at[0, 0]], rr, sr)

    def _wait(slot):
        nr, rr, su, re_, _, sn, sr = slots[slot][:7]
        pltpu.make_async_copy(node_hbm.at[su.at[0, 0]], nr, sn).wait()
        pltpu.make_async_copy(rel_hbm.at[re_.at[0, 0]], rr, sr).wait()

    def _wait_scatter(slot):
        ms, ss, obs = slots[slot][7], slots[slot][8], slots[slot][9]
        pltpu.make_async_copy(ms, acc_sh.at[obs.at[0, 0]], ss).wait()

    lane_dn = lax.GatherDimensionNumbers(
        offset_dims=(), collapsed_slice_dims=(0,), start_index_map=(0,))

    def _compute_scatter(c, slot):
        nr, rr, _, _, ob, _, _ = slots[slot][:7]
        ms, ss, obs = slots[slot][7], slots[slot][8], slots[slot][9]
        for g in range(GR):
            kv = mkg[pl.ds(c * CH + g * L, L)] * jnp.full((L,), D, jnp.int32)

            def _quad(q, carry):
                for u in range(4):
                    j2 = q * 4 + u
                    j = g * L + j2
                    perm = lax.broadcast(j2, (L,))
                    kgs = lax.gather(kv, perm[:, None], lane_dn,
                                     slice_sizes=(1,),
                                     mode=lax.GatherScatterMode.PROMISE_IN_BOUNDS)
                    acc_e = jnp.zeros((L,), jnp.float32)
                    acc_o = jnp.zeros((L,), jnp.float32)
                    for m in range(4):
                        a_ev, a_od = plsc.unpack(
                            plsc.bitcast(nr[j, pl.ds(4 * L + m * L, L)],
                                         jnp.bfloat16),
                            format=plsc.PackFormat.INTERLEAVED)
                        b_ev, b_od = plsc.unpack(
                            plsc.bitcast(rr[j, pl.ds(4 * L + m * L, L)],
                                         jnp.bfloat16),
                            format=plsc.PackFormat.INTERLEAVED)
                        c_ev = plsc.load_gather(c4v, [kgs + kconst[2 * m]])
                        c_od = plsc.load_gather(c4v, [kgs + kconst[2 * m + 1]])
                        acc_e = acc_e + jnp.maximum(a_ev + b_ev + c_ev, 0.0) * wk[2 * m]
                        acc_o = acc_o + jnp.maximum(a_od + b_od + c_od, 0.0) * wk[2 * m + 1]
                    sv = _lane_sum(acc_e + acc_o) + b0
                    av = 1.0 / (1.0 + jnp.exp(-sv))
                    for m in range(4):
                        h_ev, h_od = plsc.unpack(
                            plsc.bitcast(nr[j, pl.ds(m * L, L)],
                                         jnp.bfloat16),
                            format=plsc.PackFormat.INTERLEAVED)
                        r_ev, r_od = plsc.unpack(
                            plsc.bitcast(rr[j, pl.ds(m * L, L)],
                                         jnp.bfloat16),
                            format=plsc.PackFormat.INTERLEAVED)
                        ms[j, pl.ds(m * 2 * L, L)] = (h_ev + r_ev) * av
                        ms[j, pl.ds(m * 2 * L + L, L)] = (h_od + r_od) * av
                return carry

            lax.fori_loop(0, 4, _quad, 0)
        for g in range(GR):
            obs[0, 0, pl.ds(g * L, L)] = ob[0, 0, pl.ds(g * L, L)]
        pltpu.async_copy(ms, acc_sh.at[obs.at[0, 0]], ss, add=True)

    def _super(s_i, carry):
        sb = ebase + s_i * SUPER
        for col, buf in ((ekg_hbm, mkg), (erl_hbm, mrl), (esb_hbm, msb),
                         (eob_hbm, mob)):
            pltpu.sync_copy(col.at[pl.ds(sb, SUPER)], buf)

        _prep(0, 0)
        _fire(0)
        _prep(1, 1)
        _fire(1)

        def _pair(c2, carry2):
            c0 = c2 * 2
            _wait(0)
            pl.when(c2 > 0)(lambda: _wait_scatter(0))
            _compute_scatter(c0, 0)
            _prep(c0 + 2, 0)
            _fire(0)
            _wait(1)
            pl.when(c2 > 0)(lambda: _wait_scatter(1))
            _compute_scatter(c0 + 1, 1)
            _prep(c0 + 3, 1)
            _fire(1)
            return carry2

        lax.fori_loop(0, NP - 1, _pair, 0)
        _wait(0)
        _wait_scatter(0)
        _compute_scatter(NCH - 2, 0)
        _wait(1)
        _wait_scatter(1)
        _compute_scatter(NCH - 1, 1)
        _wait_scatter(0)
        _wait_scatter(1)
        return carry

    lax.fori_loop(0, NSUP, _super, 0)
    plsc.subcore_barrier()

    for i in range(NZ):
        r0 = sid * RW + i * ZR
        pltpu.sync_copy(acc_sh.at[pl.ds(r0, ZR)], zbuf)
        pltpu.sync_copy(zbuf, out_hbm.at[cid, pl.ds(r0, ZR)])


def _sc_edges(node_tab, rel_tab, c4, wvec, b0vec, ecols):
    ekg, erl, esb, eob = ecols
    mesh = plsc.VectorSubcoreMesh(core_axis_name="c", subcore_axis_name="s",
                                  num_cores=NC, num_subcores=NS)
    fn = pl.kernel(
        _sc_body,
        out_type=jax.ShapeDtypeStruct((NC, N_PAD, D), jnp.float32),
        mesh=mesh,
        compiler_params=pltpu.CompilerParams(needs_layout_passes=False),
        scratch_types=[
            pltpu.VMEM_SHARED((N_PAD, D), jnp.float32),
            pltpu.VMEM((CH, D), jnp.int32),
            pltpu.VMEM((CH, D), jnp.int32),
            pltpu.VMEM((CH, D), jnp.int32),
            pltpu.VMEM((CH, D), jnp.int32),
            pltpu.VMEM((CH, D), jnp.float32),
            pltpu.VMEM((CH, D), jnp.float32),
            pltpu.VMEM((ZR, D), jnp.float32),
            pltpu.VMEM((SUPER,), jnp.int32),
            pltpu.VMEM((SUPER,), jnp.int32),
            pltpu.VMEM((SUPER,), jnp.int32),
            pltpu.VMEM((SUPER,), jnp.int32),
            pltpu.VMEM((1, 1, CH), jnp.int32),
            pltpu.VMEM((1, 1, CH), jnp.int32),
            pltpu.VMEM((1, 1, CH), jnp.int32),
            pltpu.VMEM((1, 1, CH), jnp.int32),
            pltpu.VMEM((1, 1, CH), jnp.int32),
            pltpu.VMEM((1, 1, CH), jnp.int32),
            pltpu.VMEM((1, 1, CH), jnp.int32),
            pltpu.VMEM((1, 1, CH), jnp.int32),
            pltpu.VMEM((4 * D,), jnp.float32),
            pltpu.VMEM((D,), jnp.float32),
            pltpu.VMEM((L,), jnp.float32),
            pltpu.SemaphoreType.DMA,
            pltpu.SemaphoreType.DMA,
            pltpu.SemaphoreType.DMA,
            pltpu.SemaphoreType.DMA,
            pltpu.SemaphoreType.DMA,
            pltpu.SemaphoreType.DMA,
        ],
    )
    return fn(node_tab, rel_tab, c4, wvec, b0vec, ekg, erl, esb, eob)


def _fin_body(p_ref, wh_ref, out_ref):
    s = p_ref[0] + p_ref[1]
    out_ref[...] = lax.dot_general(
        s, wh_ref[...], (((1,), (1,)), ((), ())),
        preferred_element_type=jnp.float32)


def _finish(partials, wh):
    nblk = 8
    rows = N_PAD // nblk
    return pl.pallas_call(
        _fin_body,
        grid=(nblk,),
        in_specs=[
            pl.BlockSpec((NC, rows, D), lambda i: (0, i, 0)),
            pl.BlockSpec((D, D), lambda i: (0, 0)),
        ],
        out_specs=pl.BlockSpec((rows, D), lambda i: (i, 0)),
        out_shape=jax.ShapeDtypeStruct((N_PAD, D), jnp.float32),
    )(partials, wh)


def kernel(hidden, edges, n_node, kgemb, left_num, rela_embed, Ws, Wr,
           Wkg_W, Wkg_b, walpha_W, walpha_b, Wh):
    hid_p = jnp.pad(hidden, ((0, PAD_ROWS - hidden.shape[0]), (0, 0)))
    rel_p = jnp.pad(rela_embed, ((0, PAD_ROWS - rela_embed.shape[0]), (0, 0)))
    e32 = edges.astype(jnp.int32)
    npad = E_PAD - e32.shape[0]
    objc = jnp.minimum(e32[:, 5], hidden.shape[0] - 1)
    objc = jnp.pad(objc, (0, npad), constant_values=N_NODE)
    kidx = 2 * (e32[:, 1] >= left_num).astype(jnp.int32) \
        + (e32[:, 3] >= left_num).astype(jnp.int32)
    ecols = (jnp.pad(kidx, (0, npad)), jnp.pad(e32[:, 2], (0, npad)),
             jnp.pad(e32[:, 4], (0, npad)), objc)
    wvec = walpha_W.reshape(D)[SIGMA]
    b0vec = jnp.broadcast_to(walpha_b.reshape(1), (L,)).astype(jnp.float32)

    node_tab, rel_tab = _build_tables(hid_p, rel_p, Ws, Wr)
    node_tab = lax.bitcast_convert_type(
        node_tab.reshape(PAD_ROWS, D, 2), jnp.int32)
    rel_tab = lax.bitcast_convert_type(
        rel_tab.reshape(PAD_ROWS, D, 2), jnp.int32)
    c4 = _build_c4(kgemb, Wkg_W, Wkg_b)[:4][:, SIGMA].reshape(4 * D)
    partials = _sc_edges(node_tab, rel_tab, c4, wvec, b0vec, ecols)
    return _finish(partials, Wh[:, SIGMA])[:N_NODE]


# speedup vs baseline: 1.4117x; 1.0079x over previous
import functools

import numpy as np

import jax
import jax.numpy as jnp
from jax import lax
from jax.experimental import pallas as pl
from jax.experimental.pallas import tpu as pltpu
from jax.experimental.pallas import tpu_sc as plsc

N_NODE = 10000
D = 128
L = 16
NC, NS = 2, 16
NW = NC * NS
EW = 10240
E_PAD = NW * EW
SUPER = 1280
NSUP = EW // SUPER
CH = 32
NCH = SUPER // CH
NP = NCH // 2
GR = CH // L
N_PAD = 10240
RW = N_PAD // NS
ZR = 64
NZ = RW // ZR
PAD_ROWS = 10048
SIGMA = np.concatenate([np.concatenate([np.arange(32 * m, 32 * m + 32, 2),
                                        np.arange(32 * m + 1, 32 * m + 32, 2)])
                        for m in range(4)])


def _tables_body(hid_ref, rel_ref, ws_ref, wr_ref, node_ref, relo_ref):
    h = hid_ref[...]
    r = rel_ref[...]
    node_ref[:, :D] = h.astype(jnp.bfloat16)
    node_ref[:, D:] = lax.dot_general(
        h, ws_ref[...], (((1,), (1,)), ((), ())),
        preferred_element_type=jnp.float32).astype(jnp.bfloat16)
    relo_ref[:, :D] = r.astype(jnp.bfloat16)
    relo_ref[:, D:] = lax.dot_general(
        r, wr_ref[...], (((1,), (1,)), ((), ())),
        preferred_element_type=jnp.float32).astype(jnp.bfloat16)


def _build_tables(hid_p, rel_p, ws, wr):
    nblk = 8
    rows = PAD_ROWS // nblk
    return pl.pallas_call(
        _tables_body,
        grid=(nblk,),
        in_specs=[
            pl.BlockSpec((rows, D), lambda i: (i, 0)),
            pl.BlockSpec((rows, D), lambda i: (i, 0)),
            pl.BlockSpec((D, D), lambda i: (0, 0)),
            pl.BlockSpec((D, D), lambda i: (0, 0)),
        ],
        out_specs=[
            pl.BlockSpec((rows, 2 * D), lambda i: (i, 0)),
            pl.BlockSpec((rows, 2 * D), lambda i: (i, 0)),
        ],
        out_shape=[
            jax.ShapeDtypeStruct((PAD_ROWS, 2 * D), jnp.bfloat16),
            jax.ShapeDtypeStruct((PAD_ROWS, 2 * D), jnp.bfloat16),
        ],
    )(hid_p, rel_p, ws, wr)


def _c4_body(kg_ref, wkg_ref, wb_ref, out_ref):
    kg = kg_ref[...]
    w1 = wkg_ref[:, :D]
    w2 = wkg_ref[:, D:]
    kg1 = lax.dot_general(kg, w1, (((1,), (1,)), ((), ())),
                          preferred_element_type=jnp.float32)
    kg2 = lax.dot_general(kg, w2, (((1,), (1,)), ((), ())),
                          preferred_element_type=jnp.float32)
    c = kg1[:, None, :] + kg2[None, :, :] + wb_ref[...][None, None, :]
    c = c.reshape(4, D)
    out_ref[...] = jnp.concatenate([c, jnp.zeros((4, D), jnp.float32)], axis=0)


def _build_c4(kgemb, wkg_w, wkg_b):
    return pl.pallas_call(
        _c4_body,
        out_shape=jax.ShapeDtypeStruct((8, D), jnp.float32),
    )(kgemb, wkg_w, wkg_b)


def _lane_sum(v):
    lanes = lax.iota(jnp.int32, L)
    dnums = lax.GatherDimensionNumbers(
        offset_dims=(), collapsed_slice_dims=(0,), start_index_map=(0,))
    for sh in (1, 2, 4, 8):
        perm = lax.bitwise_xor(lanes, jnp.full((L,), sh, jnp.int32))
        v = v + lax.gather(v, perm[:, None], dnums, slice_sizes=(1,),
                           mode=lax.GatherScatterMode.PROMISE_IN_BOUNDS)
    return v


def _sc_body(node_hbm, rel_hbm, c4_hbm, w_hbm, b0_hbm, ekg_hbm,
             erl_hbm, esb_hbm, eob_hbm,
             out_hbm, acc_sh, nrows0, nrows1, rrows0, rrows1, msg0, msg1,
             zbuf, mkg, mrl, msb, mob, subi0, subi1, reli0, reli1, obji0,
             obji1, obsc0, obsc1, c4v, wv, b0v, semn0, semr0, semn1, semr1,
             sems0, sems1):
    cid = lax.axis_index("c")
    sid = lax.axis_index("s")
    wid = cid * NS + sid

    pltpu.sync_copy(c4_hbm, c4v)
    pltpu.sync_copy(w_hbm, wv)
    pltpu.sync_copy(b0_hbm, b0v)

    zero = jnp.zeros((L,), jnp.float32)

    def _zrow(r, carry):
        for k in range(8):
            zbuf[r, pl.ds(k * L, L)] = zero
        return carry

    lax.fori_loop(0, ZR, _zrow, 0)
    for i in range(NZ):
        pltpu.sync_copy(zbuf, acc_sh.at[pl.ds(sid * RW + i * ZR, ZR)])
    plsc.subcore_barrier()

    wk = tuple(wv[pl.ds(k * L, L)] for k in range(8))
    b0 = b0v[...]
    kconst = tuple(lax.iota(jnp.int32, L) + jnp.full((L,), k * L, jnp.int32)
                   for k in range(8))

    ebase = wid * EW
    slots = ((nrows0, rrows0, subi0, reli0, obji0, semn0, semr0, msg0,
              sems0, obsc0),
             (nrows1, rrows1, subi1, reli1, obji1, semn1, semr1, msg1,
              sems1, obsc1))

    def _prep(c, slot):
        su, re_, ob = slots[slot][2:5]
        for g in range(GR):
            su[0, 0, pl.ds(g * L, L)] = msb[pl.ds(c * CH + g * L, L)]
            re_[0, 0, pl.ds(g * L, L)] = mrl[pl.ds(c * CH + g * L, L)]
            ob[0, 0, pl.ds(g * L, L)] = mob[pl.ds(c * CH + g * L, L)]

    def _fire(slot):
        nr, rr, su, re_, _, sn, sr = slots[slot][:7]
        pltpu.async_copy(node_hbm.at[su.at[0, 0]], nr, sn)
        pltpu.async_copy(rel_hbm.at[re_.at[0, 0]], rr, sr)

    def _wait(slot):
        nr, rr, su, re_, _, sn, sr = slots[slot][:7]
        pltpu.make_async_copy(node_hbm.at[su.at[0, 0]], nr, sn).wait()
        pltpu.make_async_copy(rel_hbm.at[re_.at[0, 0]], rr, sr).wait()

    def _wait_scatter(slot):
        ms, ss, obs = slots[slot][7], slots[slot][8], slots[slot][9]
        pltpu.make_async_copy(ms, acc_sh.at[obs.at[0, 0]], ss).wait()

    lane_dn = lax.GatherDimensionNumbers(
        offset_dims=(), collapsed_slice_dims=(0,), start_index_map=(0,))

    def _compute_scatter(c, slot):
        nr, rr, _, _, ob, _, _ = slots[slot][:7]
        ms, ss, obs = slots[slot][7], slots[slot][8], slots[slot][9]
        for g in range(GR):
            kv = mkg[pl.ds(c * CH + g * L, L)] * jnp.full((L,), D, jnp.int32)

            def _quad(q, carry):
                for u in range(4):
                    j2 = q * 4 + u
                    j = g * L + j2
                    perm = lax.broadcast(j2, (L,))
                    kgs = lax.gather(kv, perm[:, None], lane_dn,
                                     slice_sizes=(1,),
                                     mode=lax.GatherScatterMode.PROMISE_IN_BOUNDS)
                    acc_e = jnp.zeros((L,), jnp.float32)
                    acc_o = jnp.zeros((L,), jnp.float32)
                    for m in range(4):
                        a_ev, a_od = plsc.unpack(
                            plsc.bitcast(nr[j, pl.ds(4 * L + m * L, L)],
                                         jnp.bfloat16),
                            format=plsc.PackFormat.INTERLEAVED)
                        b_ev, b_od = plsc.unpack(
                            plsc.bitcast(rr[j, pl.ds(4 * L + m * L, L)],
                                         jnp.bfloat16),
                            format=plsc.PackFormat.INTERLEAVED)
                        c_ev = plsc.load_gather(c4v, [kgs + kconst[2 * m]])
                        c_od = plsc.load_gather(c4v, [kgs + kconst[2 * m + 1]])
                        acc_e = acc_e + jnp.maximum(a_ev + b_ev + c_ev, 0.0) * wk[2 * m]
                        acc_o = acc_o + jnp.maximum(a_od + b_od + c_od, 0.0) * wk[2 * m + 1]
                    sv = _lane_sum(acc_e + acc_o) + b0
                    av = 1.0 / (1.0 + jnp.exp(-sv))
                    for m in range(4):
                        h_ev, h_od = plsc.unpack(
                            plsc.bitcast(nr[j, pl.ds(m * L, L)],
                                         jnp.bfloat16),
                            format=plsc.PackFormat.INTERLEAVED)
                        r_ev, r_od = plsc.unpack(
                            plsc.bitcast(rr[j, pl.ds(m * L, L)],
                                         jnp.bfloat16),
                            format=plsc.PackFormat.INTERLEAVED)
                        ms[j, pl.ds(m * 2 * L, L)] = (h_ev + r_ev) * av
                        ms[j, pl.ds(m * 2 * L + L, L)] = (h_od + r_od) * av
                return carry

            lax.fori_loop(0, 4, _quad, 0)
        for g in range(GR):
            obs[0, 0, pl.ds(g * L, L)] = ob[0, 0, pl.ds(g * L, L)]
        pltpu.async_copy(ms, acc_sh.at[obs.at[0, 0]], ss, add=True)

    def _super(s_i, carry):
        sb = ebase + s_i * SUPER
        for col, buf in ((ekg_hbm, mkg), (erl_hbm, mrl), (esb_hbm, msb),
                         (eob_hbm, mob)):
            pltpu.sync_copy(col.at[pl.ds(sb, SUPER)], buf)

        _prep(0, 0)
        _fire(0)
        _prep(1, 1)
        _fire(1)

        def _pair(c2, carry2):
            c0 = c2 * 2
            _wait(0)
            pl.when(c2 > 0)(lambda: _wait_scatter(0))
            _compute_scatter(c0, 0)
            _prep(c0 + 2, 0)
            _fire(0)
            _wait(1)
            pl.when(c2 > 0)(lambda: _wait_scatter(1))
            _compute_scatter(c0 + 1, 1)
            _prep(c0 + 3, 1)
            _fire(1)
            return carry2

        lax.fori_loop(0, NP - 1, _pair, 0)
        _wait(0)
        _wait_scatter(0)
        _compute_scatter(NCH - 2, 0)
        _wait(1)
        _wait_scatter(1)
        _compute_scatter(NCH - 1, 1)
        _wait_scatter(0)
        _wait_scatter(1)
        return carry

    lax.fori_loop(0, NSUP, _super, 0)
    plsc.subcore_barrier()

    for i in range(NZ):
        r0 = sid * RW + i * ZR
        pltpu.sync_copy(acc_sh.at[pl.ds(r0, ZR)], out_hbm.at[cid, pl.ds(r0, ZR)])


def _sc_edges(node_tab, rel_tab, c4, wvec, b0vec, ecols):
    ekg, erl, esb, eob = ecols
    mesh = plsc.VectorSubcoreMesh(core_axis_name="c", subcore_axis_name="s",
                                  num_cores=NC, num_subcores=NS)
    fn = pl.kernel(
        _sc_body,
        out_type=jax.ShapeDtypeStruct((NC, N_PAD, D), jnp.float32),
        mesh=mesh,
        compiler_params=pltpu.CompilerParams(needs_layout_passes=False),
        scratch_types=[
            pltpu.VMEM_SHARED((N_PAD, D), jnp.float32),
            pltpu.VMEM((CH, D), jnp.int32),
            pltpu.VMEM((CH, D), jnp.int32),
            pltpu.VMEM((CH, D), jnp.int32),
            pltpu.VMEM((CH, D), jnp.int32),
            pltpu.VMEM((CH, D), jnp.float32),
            pltpu.VMEM((CH, D), jnp.float32),
            pltpu.VMEM((ZR, D), jnp.float32),
            pltpu.VMEM((SUPER,), jnp.int32),
            pltpu.VMEM((SUPER,), jnp.int32),
            pltpu.VMEM((SUPER,), jnp.int32),
            pltpu.VMEM((SUPER,), jnp.int32),
            pltpu.VMEM((1, 1, CH), jnp.int32),
            pltpu.VMEM((1, 1, CH), jnp.int32),
            pltpu.VMEM((1, 1, CH), jnp.int32),
            pltpu.VMEM((1, 1, CH), jnp.int32),
            pltpu.VMEM((1, 1, CH), jnp.int32),
            pltpu.VMEM((1, 1, CH), jnp.int32),
            pltpu.VMEM((1, 1, CH), jnp.int32),
            pltpu.VMEM((1, 1, CH), jnp.int32),
            pltpu.VMEM((4 * D,), jnp.float32),
            pltpu.VMEM((D,), jnp.float32),
            pltpu.VMEM((L,), jnp.float32),
            pltpu.SemaphoreType.DMA,
            pltpu.SemaphoreType.DMA,
            pltpu.SemaphoreType.DMA,
            pltpu.SemaphoreType.DMA,
            pltpu.SemaphoreType.DMA,
            pltpu.SemaphoreType.DMA,
        ],
    )
    return fn(node_tab, rel_tab, c4, wvec, b0vec, ekg, erl, esb, eob)


def _fin_body(p_ref, wh_ref, out_ref):
    s = p_ref[0] + p_ref[1]
    out_ref[...] = lax.dot_general(
        s, wh_ref[...], (((1,), (1,)), ((), ())),
        preferred_element_type=jnp.float32)


def _finish(partials, wh):
    nblk = 8
    rows = N_PAD // nblk
    return pl.pallas_call(
        _fin_body,
        grid=(nblk,),
        in_specs=[
            pl.BlockSpec((NC, rows, D), lambda i: (0, i, 0)),
            pl.BlockSpec((D, D), lambda i: (0, 0)),
        ],
        out_specs=pl.BlockSpec((rows, D), lambda i: (i, 0)),
        out_shape=jax.ShapeDtypeStruct((N_PAD, D), jnp.float32),
    )(partials, wh)


def kernel(hidden, edges, n_node, kgemb, left_num, rela_embed, Ws, Wr,
           Wkg_W, Wkg_b, walpha_W, walpha_b, Wh):
    hid_p = jnp.pad(hidden, ((0, PAD_ROWS - hidden.shape[0]), (0, 0)))
    rel_p = jnp.pad(rela_embed, ((0, PAD_ROWS - rela_embed.shape[0]), (0, 0)))
    e32 = edges.astype(jnp.int32)
    npad = E_PAD - e32.shape[0]
    objc = jnp.minimum(e32[:, 5], hidden.shape[0] - 1)
    objc = jnp.pad(objc, (0, npad), constant_values=N_NODE)
    kidx = 2 * (e32[:, 1] >= left_num).astype(jnp.int32) \
        + (e32[:, 3] >= left_num).astype(jnp.int32)
    ecols = (jnp.pad(kidx, (0, npad)), jnp.pad(e32[:, 2], (0, npad)),
             jnp.pad(e32[:, 4], (0, npad)), objc)
    wvec = walpha_W.reshape(D)[SIGMA]
    b0vec = jnp.broadcast_to(walpha_b.reshape(1), (L,)).astype(jnp.float32)

    node_tab, rel_tab = _build_tables(hid_p, rel_p, Ws, Wr)
    node_tab = lax.bitcast_convert_type(
        node_tab.reshape(PAD_ROWS, D, 2), jnp.int32)
    rel_tab = lax.bitcast_convert_type(
        rel_tab.reshape(PAD_ROWS, D, 2), jnp.int32)
    c4 = _build_c4(kgemb, Wkg_W, Wkg_b)[:4][:, SIGMA].reshape(4 * D)
    partials = _sc_edges(node_tab, rel_tab, c4, wvec, b0vec, ecols)
    return _finish(partials, Wh[:, SIGMA])[:N_NODE]
